# Initial kernel scaffold; baseline (speedup 1.0000x reference)
import jax
import jax.numpy as jnp
from jax.experimental import pallas as pl


def kernel(x):
    raise NotImplementedError("write your pallas kernel here")


# speedup vs baseline: 1.0014x; 1.0014x over previous
import math

import jax
import jax.numpy as jnp
from jax.experimental import pallas as pl
from jax.experimental.pallas import tpu as pltpu

_EPS = 1e-12
_EPS2 = _EPS * _EPS

_TARGET_BLOCK_BYTES = 4 << 20
_MIN_STEPS = 8


def _cdiv(a, b):
    return -(-a // b)


def _l2_kernel(x_ref, o_ref):
    xf = x_ref[...]
    if xf.dtype != jnp.float32:
        xf = xf.astype(jnp.float32)
    ss = jnp.sum(xf * xf, axis=1, keepdims=True)
    o_ref[...] = (xf * jax.lax.rsqrt(jnp.maximum(ss, _EPS2))).astype(o_ref.dtype)


def _normalize_mid(x3):
    lead, c, trail = x3.shape
    itemsize = jnp.dtype(x3.dtype).itemsize

    row_bytes = c * trail * itemsize
    if trail % 128 == 0 and row_bytes <= 2 * _TARGET_BLOCK_BYTES:
        tile_t = trail
    elif trail <= 128:
        tile_t = trail
    else:
        tile_t = max(128, min(trail // 128, _TARGET_BLOCK_BYTES // (c * itemsize * 128)) * 128)

    tile_lead = max(1, min(lead, _TARGET_BLOCK_BYTES // (c * tile_t * itemsize)))
    while tile_lead > 1 and _cdiv(lead, tile_lead) * _cdiv(trail, tile_t) < _MIN_STEPS:
        tile_lead //= 2

    grid = (_cdiv(lead, tile_lead), _cdiv(trail, tile_t))
    blk = tile_lead * c * tile_t * itemsize

    return pl.pallas_call(
        _l2_kernel,
        out_shape=jax.ShapeDtypeStruct(x3.shape, x3.dtype),
        grid=grid,
        in_specs=[pl.BlockSpec((tile_lead, c, tile_t), lambda i, j: (i, 0, j))],
        out_specs=pl.BlockSpec((tile_lead, c, tile_t), lambda i, j: (i, 0, j)),
        compiler_params=pltpu.CompilerParams(
            dimension_semantics=("parallel", "parallel"),
            vmem_limit_bytes=min(int(5 * blk) + (2 << 20), 48 << 20),
        ),
    )(x3)


def kernel(x):
    shape = x.shape
    c = shape[1]
    lead = shape[0]
    trail = math.prod(shape[2:]) if len(shape) > 2 else 1
    y3 = _normalize_mid(x.reshape(lead, c, trail))
    return y3.reshape(shape)
